# Initial kernel scaffold; baseline (speedup 1.0000x reference)
import jax
import jax.numpy as jnp
from jax.experimental import pallas as pl


def kernel(x, v, w1, w0):
    raise NotImplementedError("write your pallas kernel here")


# speedup vs baseline: 10.2370x; 10.2370x over previous
import jax
import jax.numpy as jnp
from jax import lax
from jax.experimental import pallas as pl
from jax.experimental.pallas import tpu as pltpu
from jax.experimental.pallas import tpu_sc as plsc

F = 26
H = 100000
K = 16
B = 4096

NC = 2
NS = 16
L = 16
NW = NC * NS
C = B // NW
NPAIR = (F * (F - 1)) // 2


def _ffm_body(x_hbm, vflat_hbm, w1_hbm, w0_hbm, out_hbm,
              xv, w1v, w0v, idxa, idxb, rowsa, rowsb, acc, lin, res, sem):
    cid = lax.axis_index("c")
    sid = lax.axis_index("s")
    wid = sid * NC + cid
    base = wid * C

    pltpu.sync_copy(x_hbm.at[:, pl.ds(base, C)], xv)
    pltpu.sync_copy(w1_hbm, w1v)
    pltpu.sync_copy(w0_hbm, w0v)

    zero = jnp.zeros((L,), jnp.float32)
    def _z(b, _):
        acc[b] = zero
        return 0
    lax.fori_loop(0, C, _z, 0, unroll=8)
    for k in range(C // L):
        lin[pl.ds(k * L, L)] = zero

    lane = lax.iota(jnp.int32, L)

    def _lin_f(f, _):
        for k in range(C // L):
            idx = xv[f, pl.ds(k * L, L)]
            w = plsc.load_gather(w1v, [idx])
            plsc.addupdate(lin.at[pl.ds(k * L, L)], w)
        return 0
    lax.fori_loop(0, F, _lin_f, 0)

    def _pair(q, carry):
        i, j = carry
        offa = jnp.full((L,), j * H, jnp.int32)
        offb = jnp.full((L,), i * H, jnp.int32)
        for k in range(C // L):
            xa = xv[i, pl.ds(k * L, L)]
            xb = xv[j, pl.ds(k * L, L)]
            idxa[pl.ds(k * L, L)] = xa + offa
            idxb[pl.ds(k * L, L)] = xb + offb
        ca = pltpu.async_copy(vflat_hbm.at[idxa], rowsa, sem)
        cb = pltpu.async_copy(vflat_hbm.at[idxb], rowsb, sem)
        ca.wait()
        cb.wait()

        def _mac(b, _):
            plsc.addupdate(acc.at[b], rowsa[b] * rowsb[b])
            return 0
        lax.fori_loop(0, C, _mac, 0, unroll=8)

        last = j == (F - 1)
        i = jnp.where(last, i + 1, i)
        j = jnp.where(last, i + 1, j + 1)
        return i, j

    lax.fori_loop(0, NPAIR, _pair, (jnp.int32(0), jnp.int32(1)))

    w0vec = w0v[...]
    for g in range(C // L):
        t = jnp.zeros((L,), jnp.float32)
        for m in range(L):
            s = jnp.sum(acc[g * L + m], axis=0)
            t = jnp.where(lane == m, s, t)
        z = lin[pl.ds(g * L, L)] + t + w0vec
        res[pl.ds(g * L, L)] = 1.0 / (1.0 + jnp.exp(-z))
    pltpu.sync_copy(res, out_hbm.at[pl.ds(base, C)])


def kernel(x, v, w1, w0):
    x = x.astype(jnp.int32)
    vflat = v.reshape(F * H, K)
    w1f = w1.reshape(H)
    w0v = jnp.broadcast_to(w0.astype(jnp.float32), (L,))

    mesh = plsc.VectorSubcoreMesh(core_axis_name="c", subcore_axis_name="s",
                                  num_cores=NC, num_subcores=NS)
    f = pl.kernel(
        _ffm_body,
        out_type=jax.ShapeDtypeStruct((B,), jnp.float32),
        mesh=mesh,
        compiler_params=pltpu.CompilerParams(needs_layout_passes=False,
                                             use_tc_tiling_on_sc=False),
        scratch_types=[
            pltpu.VMEM((F, C), jnp.int32),
            pltpu.VMEM((H,), jnp.float32),
            pltpu.VMEM((L,), jnp.float32),
            pltpu.VMEM((C,), jnp.int32),
            pltpu.VMEM((C,), jnp.int32),
            pltpu.VMEM((C, K), jnp.float32),
            pltpu.VMEM((C, K), jnp.float32),
            pltpu.VMEM((C, K), jnp.float32),
            pltpu.VMEM((C,), jnp.float32),
            pltpu.VMEM((C,), jnp.float32),
            pltpu.SemaphoreType.DMA,
        ],
    )
    return f(x, vflat, w1f, w0v)
